# baseline (speedup 1.0000x reference)
import jax
import jax.numpy as jnp
from jax import lax
from jax.experimental import pallas as pl
from jax.experimental.pallas import tpu as pltpu
from jax.experimental.pallas import tpu_sc as plsc

_N = 10000
_E = 320000
_D = 128
_G = 64

_NC = 2
_NS = 16
_NW = _NC * _NS
_CW = 128
_RPW = 80
_ROWS = _NW * _RPW
_RPS = _ROWS // _NS
_EPAD = _ROWS * _CW
_NACC = 10112
_RPT = _NACC // _NS
_DUMP = _N
_YSTG = 632
_YLAST = _N - (_NS - 1) * _YSTG


def _pack_rows(nelem):
  r = nelem // 128
  return r + (8 - r % 8) % 8

_f32 = jnp.float32

_mesh = plsc.VectorSubcoreMesh(
    core_axis_name="c", subcore_axis_name="s", num_cores=_NC, num_subcores=_NS)

_sc_params = pltpu.CompilerParams(use_tc_tiling_on_sc=False)


def _fill(store, nrows, ncols, val):
  def body(i, carry):
    for q in range(ncols // 16):
      store(i, q, jnp.full((16,), val, _f32))
    return carry
  lax.fori_loop(0, nrows, body, 0)


def _zero_acc(zrows, acc_sh, s):
  r0 = s * _RPT
  nfull = _RPT // _CW
  rem = _RPT - nfull * _CW
  for z in range(nfull):
    pltpu.sync_copy(zrows, acc_sh.at[pl.ds(r0 + z * _CW, _CW)])
  if rem:
    pltpu.sync_copy(zrows.at[pl.ds(0, rem)],
                    acc_sh.at[pl.ds(r0 + nfull * _CW, rem)])


def _writeout(acc_sh, out_hbm, c, s):
  pltpu.sync_copy(acc_sh.at[pl.ds(s * _RPT, _RPT)],
                  out_hbm.at[c, pl.ds(s * _RPT, _RPT)])


def _make_deg_pass():
  K = 8

  def body(dst2_hbm, out_p, dstv, ones, acc_sh, ssem):
    c = lax.axis_index("c")
    s = lax.axis_index("s")
    wid = c * _NS + s
    _fill(lambda i, q, v: ones.__setitem__((i, pl.ds(q * 16, 16)), v),
          _CW, 16, 0.0)
    _zero_acc(ones, acc_sh, s)
    _fill(lambda i, q, v: ones.__setitem__((i, pl.ds(q * 16, 16)), v),
          _CW, 16, 1.0)
    plsc.subcore_barrier()
    pltpu.sync_copy(dst2_hbm.at[pl.ds(wid * _RPW, _RPW)], dstv)

    def group(g, carry):
      descs = [pltpu.async_copy(ones, acc_sh.at[dstv.at[g * K + b]], ssem,
                                add=True) for b in range(K)]
      for d in descs:
        d.wait()
      return carry
    lax.fori_loop(0, _RPW // K, group, 0)
    plsc.subcore_barrier()
    _writeout(acc_sh, out_p, c, s)

  return pl.kernel(
      body,
      out_type=jax.ShapeDtypeStruct((_NC, _NACC, 16), _f32),
      mesh=_mesh,
      compiler_params=_sc_params,
      scratch_types=[
          pltpu.VMEM((_RPW, _CW), jnp.int32),
          pltpu.VMEM((_CW, 16), _f32),
          pltpu.VMEM_SHARED((_NACC, 16), _f32),
          pltpu.SemaphoreType.DMA,
      ])


def _make_seg_pass(H):
  K = 4

  def body(ysp_p, src2_hbm, dst2_hbm, out_p, srcv, dstv, bufs, acc_sh,
           y_sh, gsem0, gsem1, ssem0, ssem1):
    c = lax.axis_index("c")
    s = lax.axis_index("s")
    @pl.when(s < _NS - 1)
    def _():
      pltpu.sync_copy(ysp_p.at[c, pl.ds(s * _YSTG, _YSTG)],
                      y_sh.at[pl.ds(s * _YSTG, _YSTG)])
    @pl.when(s == _NS - 1)
    def _():
      pltpu.sync_copy(ysp_p.at[c, pl.ds((_NS - 1) * _YSTG, _YLAST)],
                      y_sh.at[pl.ds((_NS - 1) * _YSTG, _YLAST)])
    _fill(lambda i, q, v: bufs.__setitem__((0, i, pl.ds(q * 16, 16)), v),
          _CW, H, 0.0)
    _zero_acc(bufs.at[0], acc_sh, s)
    plsc.subcore_barrier()
    pltpu.sync_copy(src2_hbm.at[pl.ds(s * _RPS, _RPS)], srcv)
    pltpu.sync_copy(dst2_hbm.at[pl.ds(s * _RPS, _RPS)], dstv)

    gsems = (gsem0, gsem1)
    ssems = (ssem0, ssem1)

    def fire_g(g, st):
      for b in range(K):
        pltpu.async_copy(y_sh.at[srcv.at[g * K + b]], bufs.at[st * K + b],
                         gsems[st])

    def drain_g(st):
      for b in range(K):
        pltpu.make_async_copy(y_sh.at[srcv.at[b]], bufs.at[st * K + b],
                              gsems[st]).wait()

    def fire_s(g, st):
      for b in range(K):
        pltpu.async_copy(bufs.at[st * K + b], acc_sh.at[dstv.at[g * K + b]],
                         ssems[st], add=True)

    def drain_s(st):
      for b in range(K):
        pltpu.make_async_copy(bufs.at[st * K + b], acc_sh.at[pl.ds(0, _CW)],
                              ssems[st]).wait()

    npairs = _RPS // K // 2

    def piped(gg, carry):
      g0 = 2 * gg
      fire_g(g0, 0)
      fire_g(g0 + 1, 1)
      drain_g(0)
      fire_s(g0, 0)
      drain_g(1)
      fire_s(g0 + 1, 1)
      drain_s(0)
      drain_s(1)
      return carry
    lax.fori_loop(0, npairs, piped, 0)
    plsc.subcore_barrier()
    _writeout(acc_sh, out_p, c, s)

  return pl.kernel(
      body,
      out_type=jax.ShapeDtypeStruct((_NC, _NACC, H), _f32),
      mesh=_mesh,
      compiler_params=_sc_params,
      scratch_types=[
          pltpu.VMEM((_RPS, _CW), jnp.int32),
          pltpu.VMEM((_RPS, _CW), jnp.int32),
          pltpu.VMEM((2 * K, _CW, H), _f32),
          pltpu.VMEM_SHARED((_NACC, H), _f32),
          pltpu.VMEM_SHARED((_N, H), _f32),
          pltpu.SemaphoreType.DMA,
          pltpu.SemaphoreType.DMA,
          pltpu.SemaphoreType.DMA,
          pltpu.SemaphoreType.DMA,
      ])


_deg_pass = _make_deg_pass()
_seg_pass16 = _make_seg_pass(16)
_seg_pass32 = _make_seg_pass(32)


_YP16 = _pack_rows(_N * 16)
_YP32 = _pack_rows(_N * 32)
_NR16 = _N * 16 // 128
_NR32 = _N * 32 // 128


def _tc0_body(x_ref, w1_ref, xw_ref):
  xw_ref[...] = jnp.dot(x_ref[...], w1_ref[...], preferred_element_type=_f32)


_tc0 = pl.pallas_call(
    _tc0_body, out_shape=jax.ShapeDtypeStruct((_N, 32), _f32))


def _tc1_body(xw_ref, dacc_ref, y1_ref, dinv_ref, cnt_scr, ya, yb):
  cnt = dacc_ref[0] + dacc_ref[1]
  for j in range(8):
    cnt_scr[j::8, :] = cnt[:, 16 * j:16 * j + 16]
  deg = cnt_scr[: _N, 0:1] + 1.0
  dinv = lax.rsqrt(deg)
  y1 = xw_ref[...] * dinv
  ya[...] = y1[:, :16]
  yb[...] = y1[:, 16:]
  for c, ys in enumerate((ya, yb)):
    y1_ref[c, :_NR16] = jnp.concatenate(
        [ys[j::8, :] for j in range(8)], axis=1)
    y1_ref[c, _NR16:] = jnp.zeros((_YP16 - _NR16, 128), _f32)
  dinv_ref[...] = dinv


_tc1 = pl.pallas_call(
    _tc1_body,
    out_shape=(jax.ShapeDtypeStruct((_NC, _YP16, 128), _f32),
               jax.ShapeDtypeStruct((_N, 1), _f32)),
    scratch_shapes=[pltpu.VMEM((_NACC, 16), _f32),
                    pltpu.VMEM((_N, 16), _f32),
                    pltpu.VMEM((_N, 16), _f32)])


def _tc2_body(acc_ref, y1_ref, dinv_ref, b1_ref, w2_ref, y2_ref, aa, ab,
              ya, yb):
  dinv = dinv_ref[...]
  for c, asr in enumerate((aa, ab)):
    ap = acc_ref[c][:_NR16] + y1_ref[c][:_NR16]
    for j in range(8):
      asr[j::8, :] = ap[:, 16 * j:16 * j + 16]
  a = jnp.concatenate([aa[...], ab[...]], axis=1)
  h1 = jnp.maximum(a * dinv + b1_ref[...], 0.0)
  y2 = jnp.dot(h1, w2_ref[...], preferred_element_type=_f32) * dinv
  ya[...] = y2[:, :32]
  yb[...] = y2[:, 32:]
  for c, ys in enumerate((ya, yb)):
    y2_ref[c, :_NR32] = jnp.concatenate(
        [ys[j::4, :] for j in range(4)], axis=1)
    y2_ref[c, _NR32:] = jnp.zeros((_YP32 - _NR32, 128), _f32)


_tc2 = pl.pallas_call(
    _tc2_body,
    out_shape=jax.ShapeDtypeStruct((_NC, _YP32, 128), _f32),
    scratch_shapes=[pltpu.VMEM((_N, 16), _f32),
                    pltpu.VMEM((_N, 16), _f32),
                    pltpu.VMEM((_N, 32), _f32),
                    pltpu.VMEM((_N, 32), _f32)])


def _tc3_body(acc_ref, y2_ref, dinv_ref, b2_ref, batch_ref, wf_ref, bf_ref,
              out_ref, ha, hb):
  for c, hs in enumerate((ha, hb)):
    ap = acc_ref[c][:_NR32] + y2_ref[c][:_NR32]
    for j in range(4):
      hs[j::4, :] = ap[:, 32 * j:32 * j + 32]
  a = jnp.concatenate([ha[...], hb[...]], axis=1)
  h2 = jnp.maximum(a * dinv_ref[...] + b2_ref[...], 0.0)
  gid = lax.broadcasted_iota(jnp.int32, (_G, _N), 0)
  seg = (gid == batch_ref[...]).astype(_f32)
  sums = jnp.dot(seg, h2, preferred_element_type=_f32)
  cnts = jnp.sum(seg, axis=1, keepdims=True)
  pooled = sums / jnp.maximum(cnts, 1.0)
  logits = jnp.dot(pooled, wf_ref[...], preferred_element_type=_f32) + bf_ref[...]
  out_ref[...] = 1.0 / (1.0 + jnp.exp(-logits))


_tc3 = pl.pallas_call(
    _tc3_body,
    out_shape=jax.ShapeDtypeStruct((_G, 1), _f32),
    scratch_shapes=[pltpu.VMEM((_N, 32), _f32),
                    pltpu.VMEM((_N, 32), _f32)])


def kernel(x, edge_index, batch, W1, b1, W2, b2, Wf, bf):
  npad = _EPAD - _E
  src2 = jnp.concatenate(
      [edge_index[0].astype(jnp.int32),
       jnp.zeros((npad,), jnp.int32)]).reshape(_ROWS, _CW)
  dst2 = jnp.concatenate(
      [edge_index[1].astype(jnp.int32),
       jnp.full((npad,), _DUMP, jnp.int32)]).reshape(_ROWS, _CW)
  dacc = _deg_pass(dst2)
  dacc_p = dacc.reshape(_NC, _NACC * 16 // 128, 128)
  xw = _tc0(x, W1)
  y1p, dinv = _tc1(xw, dacc_p)
  y1s = y1p.reshape(_NC, _YP16 * 128 // 16, 16)
  acc1 = _seg_pass16(y1s, src2, dst2)
  acc1_p = acc1.reshape(_NC, _NACC * 16 // 128, 128)
  y2p = _tc2(acc1_p, y1p, dinv, b1.reshape(1, 32), W2)
  y2s = y2p.reshape(_NC, _YP32 * 128 // 32, 32)
  acc2 = _seg_pass32(y2s, src2, dst2)
  acc2_p = acc2.reshape(_NC, _NACC * 32 // 128, 128)
  out = _tc3(acc2_p, y2p, dinv, b2.reshape(1, 64),
             batch.astype(jnp.int32).reshape(1, _N), Wf, bf.reshape(1, 1))
  return out.reshape(-1)


# speedup vs baseline: 1.0390x; 1.0390x over previous
import jax
import jax.numpy as jnp
from jax import lax
from jax.experimental import pallas as pl
from jax.experimental.pallas import tpu as pltpu
from jax.experimental.pallas import tpu_sc as plsc

_N = 10000
_E = 320000
_D = 128
_G = 64

_NC = 2
_NS = 16
_NW = _NC * _NS
_CW = 128
_RPW = 80
_ROWS = _NW * _RPW
_RPS = _ROWS // _NS
_EPAD = _ROWS * _CW
_NACC = 10112
_RPT = _NACC // _NS
_DUMP = _N
_YSTG = 632
_YLAST = _N - (_NS - 1) * _YSTG


def _pack_rows(nelem):
  r = nelem // 128
  return r + (8 - r % 8) % 8

_f32 = jnp.float32

_mesh = plsc.VectorSubcoreMesh(
    core_axis_name="c", subcore_axis_name="s", num_cores=_NC, num_subcores=_NS)

_sc_params = pltpu.CompilerParams(use_tc_tiling_on_sc=False)


def _fill(store, nrows, ncols, val):
  def body(i, carry):
    for q in range(ncols // 16):
      store(i, q, jnp.full((16,), val, _f32))
    return carry
  lax.fori_loop(0, nrows, body, 0)


def _zero_acc(zrows, acc_sh, s):
  r0 = s * _RPT
  nfull = _RPT // _CW
  rem = _RPT - nfull * _CW
  for z in range(nfull):
    pltpu.sync_copy(zrows, acc_sh.at[pl.ds(r0 + z * _CW, _CW)])
  if rem:
    pltpu.sync_copy(zrows.at[pl.ds(0, rem)],
                    acc_sh.at[pl.ds(r0 + nfull * _CW, rem)])


def _writeout(acc_sh, out_hbm, c, s):
  pltpu.sync_copy(acc_sh.at[pl.ds(s * _RPT, _RPT)],
                  out_hbm.at[c, pl.ds(s * _RPT, _RPT)])


_EROWS = _E // _CW
_DST0 = _ROWS


def _tce_body(e_ref, o_ref):
  o_ref[0:_EROWS] = e_ref[0:_EROWS]
  o_ref[_EROWS:_ROWS] = jnp.zeros((_ROWS - _EROWS, _CW), jnp.int32)
  o_ref[_ROWS:_ROWS + _EROWS] = e_ref[_EROWS:]
  o_ref[_ROWS + _EROWS:] = jnp.full((_ROWS - _EROWS, _CW), _DUMP, jnp.int32)


_tce = pl.pallas_call(
    _tce_body, out_shape=jax.ShapeDtypeStruct((2 * _ROWS, _CW), jnp.int32))


def _make_deg_pass():
  K = 8

  def body(er_hbm, out_p, dstv, ones, acc_sh, ssem):
    c = lax.axis_index("c")
    s = lax.axis_index("s")
    wid = c * _NS + s
    _fill(lambda i, q, v: ones.__setitem__((i, pl.ds(q * 16, 16)), v),
          _CW, 16, 0.0)
    _zero_acc(ones, acc_sh, s)
    _fill(lambda i, q, v: ones.__setitem__((i, pl.ds(q * 16, 16)), v),
          _CW, 16, 1.0)
    plsc.subcore_barrier()
    pltpu.sync_copy(er_hbm.at[pl.ds(_DST0 + wid * _RPW, _RPW)], dstv)

    def group(g, carry):
      descs = [pltpu.async_copy(ones, acc_sh.at[dstv.at[g * K + b]], ssem,
                                add=True) for b in range(K)]
      for d in descs:
        d.wait()
      return carry
    lax.fori_loop(0, _RPW // K, group, 0)
    plsc.subcore_barrier()
    _writeout(acc_sh, out_p, c, s)

  return pl.kernel(
      body,
      out_type=jax.ShapeDtypeStruct((_NC, _NACC, 16), _f32),
      mesh=_mesh,
      compiler_params=_sc_params,
      scratch_types=[
          pltpu.VMEM((_RPW, _CW), jnp.int32),
          pltpu.VMEM((_CW, 16), _f32),
          pltpu.VMEM_SHARED((_NACC, 16), _f32),
          pltpu.SemaphoreType.DMA,
      ])


def _make_seg_pass(H):
  K = 4

  def body(ysp_p, er_hbm, out_p, srcv, dstv, bufs, acc_sh,
           y_sh, gsem0, gsem1, ssem0, ssem1):
    c = lax.axis_index("c")
    s = lax.axis_index("s")
    @pl.when(s < _NS - 1)
    def _():
      pltpu.sync_copy(ysp_p.at[c, pl.ds(s * _YSTG, _YSTG)],
                      y_sh.at[pl.ds(s * _YSTG, _YSTG)])
    @pl.when(s == _NS - 1)
    def _():
      pltpu.sync_copy(ysp_p.at[c, pl.ds((_NS - 1) * _YSTG, _YLAST)],
                      y_sh.at[pl.ds((_NS - 1) * _YSTG, _YLAST)])
    _fill(lambda i, q, v: bufs.__setitem__((0, i, pl.ds(q * 16, 16)), v),
          _CW, H, 0.0)
    _zero_acc(bufs.at[0], acc_sh, s)
    plsc.subcore_barrier()
    pltpu.sync_copy(er_hbm.at[pl.ds(s * _RPS, _RPS)], srcv)
    pltpu.sync_copy(er_hbm.at[pl.ds(_DST0 + s * _RPS, _RPS)], dstv)

    gsems = (gsem0, gsem1)
    ssems = (ssem0, ssem1)

    def fire_g(g, st):
      for b in range(K):
        pltpu.async_copy(y_sh.at[srcv.at[g * K + b]], bufs.at[st * K + b],
                         gsems[st])

    def drain_g(st):
      for b in range(K):
        pltpu.make_async_copy(y_sh.at[srcv.at[b]], bufs.at[st * K + b],
                              gsems[st]).wait()

    def fire_s(g, st):
      for b in range(K):
        pltpu.async_copy(bufs.at[st * K + b], acc_sh.at[dstv.at[g * K + b]],
                         ssems[st], add=True)

    def drain_s(st):
      for b in range(K):
        pltpu.make_async_copy(bufs.at[st * K + b], acc_sh.at[pl.ds(0, _CW)],
                              ssems[st]).wait()

    npairs = _RPS // K // 2

    def piped(gg, carry):
      g0 = 2 * gg
      fire_g(g0, 0)
      fire_g(g0 + 1, 1)
      drain_g(0)
      fire_s(g0, 0)
      drain_g(1)
      fire_s(g0 + 1, 1)
      drain_s(0)
      drain_s(1)
      return carry
    lax.fori_loop(0, npairs, piped, 0)
    plsc.subcore_barrier()
    _writeout(acc_sh, out_p, c, s)

  return pl.kernel(
      body,
      out_type=jax.ShapeDtypeStruct((_NC, _NACC, H), _f32),
      mesh=_mesh,
      compiler_params=_sc_params,
      scratch_types=[
          pltpu.VMEM((_RPS, _CW), jnp.int32),
          pltpu.VMEM((_RPS, _CW), jnp.int32),
          pltpu.VMEM((2 * K, _CW, H), _f32),
          pltpu.VMEM_SHARED((_NACC, H), _f32),
          pltpu.VMEM_SHARED((_N, H), _f32),
          pltpu.SemaphoreType.DMA,
          pltpu.SemaphoreType.DMA,
          pltpu.SemaphoreType.DMA,
          pltpu.SemaphoreType.DMA,
      ])


_deg_pass = _make_deg_pass()
_seg_pass16 = _make_seg_pass(16)
_seg_pass32 = _make_seg_pass(32)


_YP16 = _pack_rows(_N * 16)
_YP32 = _pack_rows(_N * 32)
_NR16 = _N * 16 // 128
_NR32 = _N * 32 // 128


def _tc0_body(x_ref, w1_ref, xw_ref):
  xw_ref[...] = jnp.dot(x_ref[...], w1_ref[...], preferred_element_type=_f32)


_tc0 = pl.pallas_call(
    _tc0_body, out_shape=jax.ShapeDtypeStruct((_N, 32), _f32))


def _tc1_body(xw_ref, dacc_ref, y1_ref, dinv_ref, cnt_scr, ya, yb):
  cnt = dacc_ref[0] + dacc_ref[1]
  for j in range(8):
    cnt_scr[j::8, :] = cnt[:, 16 * j:16 * j + 16]
  deg = cnt_scr[: _N, 0:1] + 1.0
  dinv = lax.rsqrt(deg)
  y1 = xw_ref[...] * dinv
  ya[...] = y1[:, :16]
  yb[...] = y1[:, 16:]
  for c, ys in enumerate((ya, yb)):
    y1_ref[c, :_NR16] = jnp.concatenate(
        [ys[j::8, :] for j in range(8)], axis=1)
    y1_ref[c, _NR16:] = jnp.zeros((_YP16 - _NR16, 128), _f32)
  dinv_ref[...] = dinv


_tc1 = pl.pallas_call(
    _tc1_body,
    out_shape=(jax.ShapeDtypeStruct((_NC, _YP16, 128), _f32),
               jax.ShapeDtypeStruct((_N, 1), _f32)),
    scratch_shapes=[pltpu.VMEM((_NACC, 16), _f32),
                    pltpu.VMEM((_N, 16), _f32),
                    pltpu.VMEM((_N, 16), _f32)])


def _tc2_body(acc_ref, y1_ref, dinv_ref, b1_ref, w2_ref, y2_ref, aa, ab,
              ya, yb):
  dinv = dinv_ref[...]
  for c, asr in enumerate((aa, ab)):
    ap = acc_ref[c][:_NR16] + y1_ref[c][:_NR16]
    for j in range(8):
      asr[j::8, :] = ap[:, 16 * j:16 * j + 16]
  a = jnp.concatenate([aa[...], ab[...]], axis=1)
  h1 = jnp.maximum(a * dinv + b1_ref[...], 0.0)
  y2 = jnp.dot(h1, w2_ref[...], preferred_element_type=_f32) * dinv
  ya[...] = y2[:, :32]
  yb[...] = y2[:, 32:]
  for c, ys in enumerate((ya, yb)):
    y2_ref[c, :_NR32] = jnp.concatenate(
        [ys[j::4, :] for j in range(4)], axis=1)
    y2_ref[c, _NR32:] = jnp.zeros((_YP32 - _NR32, 128), _f32)


_tc2 = pl.pallas_call(
    _tc2_body,
    out_shape=jax.ShapeDtypeStruct((_NC, _YP32, 128), _f32),
    scratch_shapes=[pltpu.VMEM((_N, 16), _f32),
                    pltpu.VMEM((_N, 16), _f32),
                    pltpu.VMEM((_N, 32), _f32),
                    pltpu.VMEM((_N, 32), _f32)])


def _tc3_body(acc_ref, y2_ref, dinv_ref, b2_ref, batch_ref, wf_ref, bf_ref,
              out_ref, ha, hb):
  for c, hs in enumerate((ha, hb)):
    ap = acc_ref[c][:_NR32] + y2_ref[c][:_NR32]
    for j in range(4):
      hs[j::4, :] = ap[:, 32 * j:32 * j + 32]
  a = jnp.concatenate([ha[...], hb[...]], axis=1)
  h2 = jnp.maximum(a * dinv_ref[...] + b2_ref[...], 0.0)
  gid = lax.broadcasted_iota(jnp.int32, (_G, _N), 0)
  seg = (gid == batch_ref[...]).astype(_f32)
  sums = jnp.dot(seg, h2, preferred_element_type=_f32)
  cnts = jnp.sum(seg, axis=1, keepdims=True)
  pooled = sums / jnp.maximum(cnts, 1.0)
  logits = jnp.dot(pooled, wf_ref[...], preferred_element_type=_f32) + bf_ref[...]
  out_ref[...] = 1.0 / (1.0 + jnp.exp(-logits))


_tc3 = pl.pallas_call(
    _tc3_body,
    out_shape=jax.ShapeDtypeStruct((_G, 1), _f32),
    scratch_shapes=[pltpu.VMEM((_N, 32), _f32),
                    pltpu.VMEM((_N, 32), _f32)])


def kernel(x, edge_index, batch, W1, b1, W2, b2, Wf, bf):
  er = _tce(edge_index.astype(jnp.int32).reshape(2 * _EROWS, _CW))
  dacc = _deg_pass(er)
  dacc_p = dacc.reshape(_NC, _NACC * 16 // 128, 128)
  xw = _tc0(x, W1)
  y1p, dinv = _tc1(xw, dacc_p)
  y1s = y1p.reshape(_NC, _YP16 * 128 // 16, 16)
  acc1 = _seg_pass16(y1s, er)
  acc1_p = acc1.reshape(_NC, _NACC * 16 // 128, 128)
  y2p = _tc2(acc1_p, y1p, dinv, b1.reshape(1, 32), W2)
  y2s = y2p.reshape(_NC, _YP32 * 128 // 32, 32)
  acc2 = _seg_pass32(y2s, er)
  acc2_p = acc2.reshape(_NC, _NACC * 32 // 128, 128)
  out = _tc3(acc2_p, y2p, dinv, b2.reshape(1, 64),
             batch.astype(jnp.int32).reshape(1, _N), Wf, bf.reshape(1, 1))
  return out.reshape(-1)
